# baseline (speedup 1.0000x reference)
import functools

import jax
import jax.numpy as jnp
import numpy as np
from jax import lax
from jax.experimental import pallas as pl
from jax.experimental.pallas import tpu as pltpu
from jax.experimental.pallas import tpu_sc as plsc

_K = 8
_NEG_INF = np.float32(-np.inf)

_ERFC_P = [
    2.326819970068386e-2, -1.387039388740657e-1, 3.687424674597105e-1,
    -5.824733027278666e-1, 6.210004621745983e-1, -4.944515323274145e-1,
    3.404879937665872e-1, -2.741127028184656e-1, 5.638259427386472e-1,
]
_ERFC_R = [
    -1.047766399936249e+1, 1.297719955372516e+1, -7.495518717768503e+0,
    2.921019019210786e+0, -1.015265279202700e+0, 4.218463358204948e-1,
    -2.820767439740514e-1, 5.641895067754075e-1,
]
_ERF_T = [
    7.853861353153693e-5, -8.010193625184903e-4, 5.188327685732524e-3,
    -2.685381193529856e-2, 1.128358514861418e-1, -3.761262582423300e-1,
    1.128379165726710e+0,
]
_MAXLOG = np.float32(88.72283905206835)


def _poly(x, coeffs):
    p = jnp.zeros_like(x)
    for c in coeffs:
        p = p * x + np.float32(c)
    return p


def _erfc(x):
    abs_x = jnp.abs(x)
    z = jnp.exp(-x * x)
    q = np.float32(1.0) / abs_x
    yq = q * q
    p = jnp.where(abs_x < np.float32(2.0), _poly(yq, _ERFC_P), _poly(yq, _ERFC_R))
    yv = z * q * p
    y_clamp = jnp.where(-x * x < -_MAXLOG, np.float32(0.0), yv)
    erfc_big = jnp.where(x < np.float32(0.0), np.float32(2.0) - y_clamp, y_clamp)
    erf_small = x * _poly(x * x, _ERF_T)
    return jnp.where(abs_x > np.float32(1.0), erfc_big, np.float32(1.0) - erf_small)


def _gelu(x):
    return np.float32(0.5) * x * _erfc(-x * np.float32(np.sqrt(0.5)))


def _conv_body(x_ref, wq_ref, w2_ref, cb_ref, y_ref):
    xb = x_ref[...]
    bt, c, d = xb.shape
    cp = c // 2
    z = jnp.zeros((bt, 1, d), jnp.float32)
    xm1 = jnp.concatenate([z, xb[:, :-1, :]], axis=1)
    PA = jnp.dot(xb.reshape(bt * cp, 2 * d), wq_ref[...],
                 preferred_element_type=jnp.float32).reshape(bt, cp, 2)
    PB = jnp.dot(xm1.reshape(bt * cp, 2 * d), wq_ref[...],
                 preferred_element_type=jnp.float32).reshape(bt, cp, 2)
    pe = jnp.dot(xb[:, c - 1, :], w2_ref[...],
                 preferred_element_type=jnp.float32)
    z1 = jnp.zeros((bt, 1), jnp.float32)
    cb = cb_ref[0, 0]
    y_odd = (PA[:, :, 0] + jnp.concatenate([PA[:, 1:, 1], z1], axis=1)) + cb
    y_even = (PB[:, :, 0] + jnp.concatenate([PB[:, 1:, 1], pe], axis=1)) + cb
    y_ref[...] = jnp.stack([y_even, y_odd], axis=2).reshape(bt, c)


def _mlp_body(y_ref, w1_ref, b1_ref, w2_ref, b2_ref, logit_ref):
    ni = pl.program_id(1)
    nn = pl.num_programs(1)
    y = y_ref[...]
    h = jnp.dot(y, w1_ref[...], preferred_element_type=jnp.float32)
    h = h + b1_ref[0, :][None, :]
    g = _gelu(h)
    partial = jnp.dot(g, w2_ref[...], preferred_element_type=jnp.float32)

    @pl.when(ni == 0)
    def _():
        logit_ref[...] = partial

    @pl.when(ni != 0)
    def _():
        logit_ref[...] = logit_ref[...] + partial

    @pl.when(ni == nn - 1)
    def _():
        logit_ref[...] = logit_ref[...] + b2_ref[0, :][None, :]


_GATHER_DN = lax.GatherDimensionNumbers(
    offset_dims=(), collapsed_slice_dims=(0,), start_index_map=(0,))


def _gather16(v, idx):
    return lax.gather(v, idx.reshape(16, 1), _GATHER_DN, slice_sizes=(1,),
                      mode=lax.GatherScatterMode.PROMISE_IN_BOUNDS)


def _topk_sc_body(logit_hbm, val_hbm, idx_hbm, lv, rv, ri):
    nc = 2
    wid = lax.axis_index("s") * nc + lax.axis_index("c")
    rows = 64
    base = wid * rows
    pltpu.sync_copy(logit_hbm.at[pl.ds(base, rows)], lv)
    lane = lax.iota(jnp.int32, 16)

    def _splat_max(v):
        for s in (8, 4, 2, 1):
            v = jnp.maximum(v, _gather16(v, jnp.bitwise_xor(lane, s)))
        return v

    def _splat_min_i32(v):
        for s in (8, 4, 2, 1):
            v = jnp.minimum(v, _gather16(v, jnp.bitwise_xor(lane, s)))
        return v

    def _halfsum(v):
        for s in (4, 2, 1):
            v = v + _gather16(v, jnp.bitwise_xor(lane, s))
        return v

    def pair_body(pr, carry):
        res_v = jnp.zeros((16,), jnp.float32)
        res_i = jnp.zeros((16,), jnp.int32)
        for half in range(2):
            r = 2 * pr + half
            vs = [lv[r, pl.ds(16 * t, 16)] for t in range(4)]
            idxs = [lane + 16 * t for t in range(4)]
            offs = 8 * half
            for k in range(_K):
                m01 = jnp.maximum(vs[0], vs[1])
                m23 = jnp.maximum(vs[2], vs[3])
                mvec = _splat_max(jnp.maximum(m01, m23))
                cands = [jnp.where(vs[t] == mvec, idxs[t], 64) for t in range(4)]
                cmin = jnp.minimum(jnp.minimum(cands[0], cands[1]),
                                   jnp.minimum(cands[2], cands[3]))
                amvec = _splat_min_i32(cmin)
                sel = lane == (offs + k)
                res_v = jnp.where(sel, mvec, res_v)
                res_i = jnp.where(sel, amvec, res_i)
                vs = [jnp.where(idxs[t] == amvec, _NEG_INF, vs[t])
                      for t in range(4)]
        low = lane < 8
        mxv = _gather16(res_v, jnp.where(low, 0, 8))
        ev = jnp.exp(res_v - mxv)
        sv = _halfsum(ev)
        rv[pl.ds(pr * 16, 16)] = ev / sv
        ri[pl.ds(pr * 16, 16)] = res_i
        return carry

    lax.fori_loop(0, rows // 2, pair_body, 0)
    pltpu.sync_copy(rv, val_hbm.at[pl.ds(base * _K, rows * _K)])
    pltpu.sync_copy(ri, idx_hbm.at[pl.ds(base * _K, rows * _K)])


def kernel(x, conv_w, conv_b, W1, b1, W2, b2):
    B, C, D = x.shape
    E = W2.shape[1]
    w0 = conv_w[0, :, 0]
    w1c = conv_w[0, :, 1]
    w2 = conv_w[0, :, 2]
    zz = jnp.zeros((D,), jnp.float32)
    wq = jnp.stack([jnp.concatenate([w0, w1c]), jnp.concatenate([w2, zz])], axis=1)
    w2r = w2.reshape(D, 1)
    cb = conv_b.reshape(1, 1)
    b1r = b1.reshape(1, -1)
    b2r = b2.reshape(1, -1)

    bt_a = 16
    y = pl.pallas_call(
        _conv_body,
        grid=(B // bt_a,),
        in_specs=[
            pl.BlockSpec((bt_a, C, D), lambda bi: (bi, 0, 0)),
            pl.BlockSpec((2 * D, 2), lambda bi: (0, 0)),
            pl.BlockSpec((D, 1), lambda bi: (0, 0)),
            pl.BlockSpec((1, 1), lambda bi: (0, 0)),
        ],
        out_specs=pl.BlockSpec((bt_a, C), lambda bi: (bi, 0)),
        out_shape=jax.ShapeDtypeStruct((B, C), jnp.float32),
    )(x, wq, w2r, cb)

    bt_b = min(256, B)
    n1 = W1.shape[1]
    nchunk = 4 if n1 % 4 == 0 else 1
    nt = n1 // nchunk
    logits = pl.pallas_call(
        _mlp_body,
        grid=(B // bt_b, nchunk),
        in_specs=[
            pl.BlockSpec((bt_b, C), lambda bi, ni: (bi, 0)),
            pl.BlockSpec((C, nt), lambda bi, ni: (0, ni)),
            pl.BlockSpec((1, nt), lambda bi, ni: (0, ni)),
            pl.BlockSpec((nt, E), lambda bi, ni: (ni, 0)),
            pl.BlockSpec((1, E), lambda bi, ni: (0, 0)),
        ],
        out_specs=pl.BlockSpec((bt_b, E), lambda bi, ni: (bi, 0)),
        out_shape=jax.ShapeDtypeStruct((B, E), jnp.float32),
    )(y, W1, b1r, W2, b2r)

    mesh = plsc.VectorSubcoreMesh(core_axis_name="c", subcore_axis_name="s")
    topk = functools.partial(
        pl.kernel,
        mesh=mesh,
        out_type=[
            jax.ShapeDtypeStruct((B * _K,), jnp.float32),
            jax.ShapeDtypeStruct((B * _K,), jnp.int32),
        ],
        scratch_types=[
            pltpu.VMEM((64, E), jnp.float32),
            pltpu.VMEM((64 * _K,), jnp.float32),
            pltpu.VMEM((64 * _K,), jnp.int32),
        ],
    )(_topk_sc_body)
    val_flat, idx_flat = topk(logits)
    return (val_flat.reshape(B, _K), idx_flat.reshape(B, _K))
